# baseline (speedup 1.0000x reference)
import functools

import jax
import jax.numpy as jnp
from jax import lax
from jax.experimental import pallas as pl
from jax.experimental.pallas import tpu as pltpu
from jax.experimental.pallas import tpu_sc as plsc

D = 384
NC = 2
NS = 16
NW = NC * NS
CH = 128


@functools.lru_cache(maxsize=None)
def _build(batch, seq):
    B = batch * seq
    bpw = B // NW
    nch = bpw // CH
    wpr = seq // bpw
    mesh = plsc.VectorSubcoreMesh(core_axis_name="c", subcore_axis_name="s")

    @functools.partial(
        pl.kernel,
        out_type=[jax.ShapeDtypeStruct((B, D), jnp.float32)] * 6,
        mesh=mesh,
        scratch_types=[
            pltpu.VMEM((nch, CH), jnp.int32),
            pltpu.VMEM((2, CH, D), jnp.float32),
            pltpu.SemaphoreType.DMA,
            pltpu.SemaphoreType.DMA,
            pltpu.SemaphoreType.DMA,
            pltpu.SemaphoreType.DMA,
        ],
    )
    def emb3(idx_hbm, t0, t1, t2, o0, o1, o2, o3, o4, o5, idx_v, rows_v,
             g0, g1, w0, w1):
        wid = lax.axis_index("s") * NC + lax.axis_index("c")
        base = wid * bpw
        row = wid // wpr
        col = (wid % wpr) * bpw
        for j in range(nch):
            pltpu.sync_copy(idx_hbm.at[row, pl.ds(col + j * CH, CH)], idx_v.at[j])
        gsem = (g0, g1)
        wsem = (w0, w1)
        units = [
            (tbl, outa, outb, j)
            for j in range(nch)
            for tbl, outa, outb in ((t0, o0, o3), (t1, o1, o4), (t2, o2, o5))
        ]
        n = len(units)
        g = [None] * n
        w = [None] * n

        def fire_writes(u):
            tbl, outa, outb, j = units[u]
            slot = u % 2
            dst = pl.ds(base + j * CH, CH)
            return (
                pltpu.async_copy(rows_v.at[slot], outa.at[dst], wsem[slot]),
                pltpu.async_copy(rows_v.at[slot], outb.at[dst], wsem[slot]),
            )

        for u, (tbl, outa, outb, j) in enumerate(units):
            slot = u % 2
            if u >= 2:
                w[u - 2][0].wait()
                w[u - 2][1].wait()
            g[u] = pltpu.async_copy(tbl.at[idx_v.at[j]], rows_v.at[slot], gsem[slot])
            if u >= 1:
                g[u - 1].wait()
                w[u - 1] = fire_writes(u - 1)
        g[n - 1].wait()
        w[n - 1] = fire_writes(n - 1)
        for d in w[n - 2]:
            d.wait()
        for d in w[n - 1]:
            d.wait()

    return emb3


def kernel(inputs, emb0, emb1, emb2):
    batch, seq = inputs.shape
    outs = _build(batch, seq)(inputs.astype(jnp.int32), emb0, emb1, emb2)
    return tuple(o.reshape(batch, seq, D) for o in outs)


# speedup vs baseline: 1.0064x; 1.0064x over previous
import functools

import jax
import jax.numpy as jnp
from jax import lax
from jax.experimental import pallas as pl
from jax.experimental.pallas import tpu as pltpu
from jax.experimental.pallas import tpu_sc as plsc

D = 384
NC = 2
NS = 16
NW = NC * NS
CH = 128


@functools.lru_cache(maxsize=None)
def _build(batch, seq):
    B = batch * seq
    bpw = B // NW
    nch = bpw // CH
    wpr = seq // bpw
    mesh = plsc.VectorSubcoreMesh(core_axis_name="c", subcore_axis_name="s")

    first = [(0, 32), (32, 96)] + [(j, CH) for j in range(CH, bpw, CH)]
    rest = [(j, CH) for j in range(0, bpw, CH)]

    @functools.partial(
        pl.kernel,
        out_type=[jax.ShapeDtypeStruct((B, D), jnp.float32)] * 6,
        mesh=mesh,
        scratch_types=[
            pltpu.VMEM((bpw,), jnp.int32),
            pltpu.VMEM((2, CH, D), jnp.float32),
            pltpu.SemaphoreType.DMA,
            pltpu.SemaphoreType.DMA,
            pltpu.SemaphoreType.DMA,
            pltpu.SemaphoreType.DMA,
        ],
    )
    def emb3(idx_hbm, t0, t1, t2, o0, o1, o2, o3, o4, o5, idx_v, rows_v,
             g0, g1, w0, w1):
        wid = lax.axis_index("s") * NC + lax.axis_index("c")
        base = wid * bpw
        row = wid // wpr
        col = (wid % wpr) * bpw
        pltpu.sync_copy(idx_hbm.at[row, pl.ds(col, bpw)], idx_v)
        gsem = (g0, g1)
        wsem = (w0, w1)
        units = [
            (tbl, outa, outb, off, ln)
            for k, (tbl, outa, outb) in enumerate(
                ((t0, o0, o3), (t1, o1, o4), (t2, o2, o5)))
            for off, ln in (first if k == 0 else rest)
        ]
        n = len(units)
        g = [None] * n
        w = [None] * n

        def fire_writes(u):
            tbl, outa, outb, off, ln = units[u]
            slot = u % 2
            src = rows_v.at[slot, pl.ds(0, ln)]
            dst = pl.ds(base + off, ln)
            return (
                pltpu.async_copy(src, outa.at[dst], wsem[slot]),
                pltpu.async_copy(src, outb.at[dst], wsem[slot]),
            )

        for u, (tbl, outa, outb, off, ln) in enumerate(units):
            slot = u % 2
            if u >= 2:
                w[u - 2][0].wait()
                w[u - 2][1].wait()
            g[u] = pltpu.async_copy(tbl.at[idx_v.at[pl.ds(off, ln)]],
                                    rows_v.at[slot, pl.ds(0, ln)], gsem[slot])
            if u >= 1:
                g[u - 1].wait()
                w[u - 1] = fire_writes(u - 1)
        g[n - 1].wait()
        w[n - 1] = fire_writes(n - 1)
        for d in w[n - 2]:
            d.wait()
        for d in w[n - 1]:
            d.wait()

    return emb3


def kernel(inputs, emb0, emb1, emb2):
    batch, seq = inputs.shape
    outs = _build(batch, seq)(inputs.astype(jnp.int32), emb0, emb1, emb2)
    return tuple(o.reshape(batch, seq, D) for o in outs)
